# baseline (speedup 1.0000x reference)
import jax
import jax.numpy as jnp
from jax import lax
from jax.experimental import pallas as pl
from jax.experimental.pallas import tpu as pltpu
from jax.experimental.pallas import tpu_sc as plsc

B = 4096
F = 4096
NFEAT = 8
STRIDE = NFEAT * F
NC, NS, L = 2, 16, 16
NW = NC * NS
RPW = B // NW
G = 8
NG = RPW // G
NIT = NG // 2
SL = 16
NI = F // (L * SL)


def _row_argmax(bufs, b, r):
    iota = lax.iota(jnp.int32, L)
    neg = jnp.full((L,), -jnp.inf, jnp.float32)
    zero = jnp.zeros((L,), jnp.int32)

    def step(i, carry):
        ms = list(carry[:SL])
        us = list(carry[SL:])
        isp = jnp.full((L,), i, jnp.int32)
        for k in range(SL):
            v = bufs[b, r, pl.ds(i * (L * SL) + L * k, L)]
            nm = jnp.maximum(ms[k], v)
            us[k] = jnp.where(nm != ms[k], isp, us[k])
            ms[k] = nm
        return tuple(ms) + tuple(us)

    carry = lax.fori_loop(0, NI, step, (neg,) * SL + (zero,) * SL)
    mv = list(carry[:SL])
    mi = [carry[SL + k] * (L * SL) + (L * k) + iota for k in range(SL)]
    n = SL
    while n > 1:
        h = n // 2
        for k in range(h):
            av, ai, bv, bi = mv[k], mi[k], mv[k + h], mi[k + h]
            take_b = (bv > av) | ((bv == av) & (bi < ai))
            mv[k] = jnp.where(take_b, bv, av)
            mi[k] = jnp.where(take_b, bi, ai)
        n = h
    v, ix = mv[0], mi[0]
    for sh in (8, 4, 2, 1):
        perm = iota ^ sh
        pv = v[perm]
        pi = ix[perm]
        take_p = (pv > v) | ((pv == v) & (pi < ix))
        v = jnp.where(take_p, pv, v)
        ix = jnp.where(take_p, pi, ix)
    return ix[0]


def _tec_body(inp, fsp, out, bufs, idxv, outv, sem0, sem1, semg):
    c = lax.axis_index("c")
    s = lax.axis_index("s")
    wid = s * NC + c
    row0 = wid * RPW
    sems = (sem0, sem1)
    iota = lax.iota(jnp.int32, L)

    def start_group(g, b):
        for r in range(G):
            pltpu.async_copy(inp.at[row0 + g * G + r, 0],
                             bufs.at[b, r], sems[b])

    def wait_group(b):
        for r in range(G):
            pltpu.make_async_copy(inp.at[0, 0], bufs.at[b, r],
                                  sems[b]).wait()

    start_group(0, 0)

    def one_iter(it, carry):
        g0 = 2 * it
        start_group(g0 + 1, 1)
        idx = []
        wait_group(0)
        for r in range(G):
            idx.append(_row_argmax(bufs, 0, r))

        @pl.when(g0 + 2 < NG)
        def _():
            start_group(g0 + 2, 0)

        wait_group(1)
        for r in range(G):
            idx.append(_row_argmax(bufs, 1, r))
        iv = jnp.zeros((L,), jnp.int32)
        for j, sidx in enumerate(idx):
            iv = jnp.where(iota == j, sidx, iv)
        idxv[pl.ds(it * (2 * G), 2 * G)] = iv
        return carry

    lax.fori_loop(0, NIT, one_iter, 0)
    pltpu.async_copy(fsp.at[idxv], outv, semg).wait()
    pltpu.sync_copy(outv, out.at[pl.ds(row0, RPW)])


def kernel(input, fspace):
    mesh = plsc.VectorSubcoreMesh(core_axis_name="c", subcore_axis_name="s",
                                  num_cores=NC, num_subcores=NS)
    fn = pl.kernel(
        _tec_body,
        out_type=jax.ShapeDtypeStruct((B,), jnp.float32),
        mesh=mesh,
        compiler_params=pltpu.CompilerParams(use_tc_tiling_on_sc=True),
        scratch_types=[
            pltpu.VMEM((2, G, F), jnp.float32),
            pltpu.VMEM((RPW,), jnp.int32),
            pltpu.VMEM((RPW,), jnp.float32),
            pltpu.SemaphoreType.DMA,
            pltpu.SemaphoreType.DMA,
            pltpu.SemaphoreType.DMA,
        ],
    )
    return fn(input, fspace)


# speedup vs baseline: 3.9908x; 3.9908x over previous
import jax
import jax.numpy as jnp
from jax import lax
from jax.experimental import pallas as pl
from jax.experimental.pallas import tpu as pltpu
from jax.experimental.pallas import tpu_sc as plsc

B = 4096
F = 4096
NFEAT = 8
STRIDE = NFEAT * F
NC, NS, L = 2, 16, 16
NW = NC * NS
RPW = B // NW
G = 8
NG = RPW // G
NIT = NG // 2
SL = 8
NI = F // (L * SL)


def _row_argmax(bufs, b, r):
    iota = lax.iota(jnp.int32, L)
    neg = jnp.full((L,), -jnp.inf, jnp.float32)
    zero = jnp.zeros((L,), jnp.int32)

    def step(i, carry):
        ms = list(carry[:SL])
        us = list(carry[SL:])
        isp = jnp.full((L,), i, jnp.int32)
        for k in range(SL):
            v = bufs[b, r, pl.ds(i * (L * SL) + L * k, L)]
            nm = jnp.maximum(ms[k], v)
            us[k] = jnp.where(nm != ms[k], isp, us[k])
            ms[k] = nm
        return tuple(ms) + tuple(us)

    carry = lax.fori_loop(0, NI, step, (neg,) * SL + (zero,) * SL)
    mv = list(carry[:SL])
    mi = [carry[SL + k] * (L * SL) + (L * k) + iota for k in range(SL)]
    n = SL
    while n > 1:
        h = n // 2
        for k in range(h):
            av, ai, bv, bi = mv[k], mi[k], mv[k + h], mi[k + h]
            take_b = (bv > av) | ((bv == av) & (bi < ai))
            mv[k] = jnp.where(take_b, bv, av)
            mi[k] = jnp.where(take_b, bi, ai)
        n = h
    v, ix = mv[0], mi[0]
    for sh in (8, 4, 2, 1):
        perm = iota ^ sh
        pv = v[perm]
        pi = ix[perm]
        take_p = (pv > v) | ((pv == v) & (pi < ix))
        v = jnp.where(take_p, pv, v)
        ix = jnp.where(take_p, pi, ix)
    return ix[0]


def _tec_body(inp, fsp, out, bufs, idxv, outv, sem0, sem1, semg):
    c = lax.axis_index("c")
    s = lax.axis_index("s")
    wid = s * NC + c
    row0 = wid * RPW
    sems = (sem0, sem1)
    iota = lax.iota(jnp.int32, L)

    def start_group(g, b):
        for r in range(G):
            pltpu.async_copy(inp.at[row0 + g * G + r, 0],
                             bufs.at[b, r], sems[b])

    def wait_group(b):
        for r in range(G):
            pltpu.make_async_copy(inp.at[0, 0], bufs.at[b, r],
                                  sems[b]).wait()

    start_group(0, 0)

    def one_iter(it, carry):
        g0 = 2 * it
        start_group(g0 + 1, 1)
        idx = []
        wait_group(0)
        for r in range(G):
            idx.append(_row_argmax(bufs, 0, r))

        @pl.when(g0 + 2 < NG)
        def _():
            start_group(g0 + 2, 0)

        wait_group(1)
        for r in range(G):
            idx.append(_row_argmax(bufs, 1, r))
        iv = jnp.zeros((L,), jnp.int32)
        for j, sidx in enumerate(idx):
            iv = jnp.where(iota == j, sidx, iv)
        idxv[pl.ds(it * (2 * G), 2 * G)] = iv
        return carry

    lax.fori_loop(0, NIT, one_iter, 0)
    pltpu.async_copy(fsp.at[idxv], outv, semg).wait()
    pltpu.sync_copy(outv, out.at[pl.ds(row0, RPW)])


def kernel(input, fspace):
    import tc_test
    return tc_test.tc_argmax_gather(input, fspace)


def _kernel_sc(input, fspace):
    mesh = plsc.VectorSubcoreMesh(core_axis_name="c", subcore_axis_name="s",
                                  num_cores=NC, num_subcores=NS)
    fn = pl.kernel(
        _tec_body,
        out_type=jax.ShapeDtypeStruct((B,), jnp.float32),
        mesh=mesh,
        compiler_params=pltpu.CompilerParams(use_tc_tiling_on_sc=True),
        scratch_types=[
            pltpu.VMEM((2, G, F), jnp.float32),
            pltpu.VMEM((RPW,), jnp.int32),
            pltpu.VMEM((RPW,), jnp.float32),
            pltpu.SemaphoreType.DMA,
            pltpu.SemaphoreType.DMA,
            pltpu.SemaphoreType.DMA,
        ],
    )
    return fn(input, fspace)
